# baseline (speedup 1.0000x reference)
import dataclasses

import jax
import jax.numpy as jnp
from jax import lax
from jax.experimental import pallas as pl
from jax.experimental.pallas import tpu as pltpu
from jax.experimental.pallas import tpu_sc as plsc

N = 10000
E = 320000
H = 128
NUM_INV = 16

NC = 2
NS = 16
NW = NC * NS
CHUNK = 64
NPAD = 10240
EPAD = 327680
CHUNKS_PER_W = EPAD // (NW * CHUNK)
C0 = 140
C1 = 2 * CHUNKS_PER_W - C0
EDGES_PER_W = EPAD // NW
HALF = N // 2
HR = 5008
TRASH = HALF
RPS = 312
RTAIL = HALF - NS * RPS


def _proj_body(xs_ref, xr_ref, wa_ref, wb_ref, p_ref, q_ref):
    dn = (((1,), (0,)), ((), ()))
    p_ref[...] = lax.dot_general(xs_ref[...], wa_ref[...], dn,
                                 precision=lax.Precision.HIGHEST,
                                 preferred_element_type=jnp.float32)
    q_ref[...] = lax.dot_general(xr_ref[...], wb_ref[...], dn,
                                 precision=lax.Precision.HIGHEST,
                                 preferred_element_type=jnp.float32)


def _edge_bias_body(ea_ref, wc_ref, b1_ref, eb_ref):
    dn = (((1,), (0,)), ((), ()))
    eb_ref[...] = lax.dot_general(ea_ref[...], wc_ref[...], dn,
                                  preferred_element_type=jnp.float32) + b1_ref[...]


def _final_body(xr_ref, p0_ref, p1_ref, o_ref):
    o_ref[...] = xr_ref[...] + p0_ref[...] + p1_ref[...]


def _node_proj(xs_p, xr_p, w1a, w1b):
    blk = 1024
    return pl.pallas_call(
        _proj_body,
        grid=(NPAD // blk,),
        in_specs=[
            pl.BlockSpec((blk, H), lambda i: (i, 0)),
            pl.BlockSpec((blk, H), lambda i: (i, 0)),
            pl.BlockSpec((H, H), lambda i: (0, 0)),
            pl.BlockSpec((H, H), lambda i: (0, 0)),
        ],
        out_specs=[
            pl.BlockSpec((blk, H), lambda i: (i, 0)),
            pl.BlockSpec((blk, H), lambda i: (i, 0)),
        ],
        out_shape=[
            jax.ShapeDtypeStruct((NPAD, H), jnp.float32),
            jax.ShapeDtypeStruct((NPAD, H), jnp.float32),
        ],
    )(xs_p, xr_p, w1a, w1b)


def _edge_bias(ea_p, w1c, b1_row):
    blk = 4096
    return pl.pallas_call(
        _edge_bias_body,
        grid=(EPAD // blk,),
        in_specs=[
            pl.BlockSpec((blk, NUM_INV), lambda i: (i, 0)),
            pl.BlockSpec((NUM_INV, H), lambda i: (0, 0)),
            pl.BlockSpec((1, H), lambda i: (0, 0)),
        ],
        out_specs=pl.BlockSpec((blk, H), lambda i: (i, 0)),
        out_shape=jax.ShapeDtypeStruct((EPAD, H), jnp.float32),
    )(ea_p, w1c, b1_row)


def _final_add(x_rec, part0, part1):
    blk = 2000
    return pl.pallas_call(
        _final_body,
        grid=(N // blk,),
        in_specs=[pl.BlockSpec((blk, H), lambda i: (i, 0))] * 3,
        out_specs=pl.BlockSpec((blk, H), lambda i: (i, 0)),
        out_shape=jax.ShapeDtypeStruct((N, H), jnp.float32),
    )(x_rec, part0, part1)


def _sc_body(p_hbm, q_hbm, eb_hbm, is_hbm, ir_hbm, z_hbm, w2_hbm, b2_hbm,
             out_hbm,
             is_v0, ir_v0, ir_lo0, ir_hi0, p_v0, q_v0, e_v0,
             is_v1, ir_v1, ir_lo1, ir_hi1, p_v1, q_v1, e_v1,
             w2_v, b2_v,
             acc_lo, acc_hi,
             gsem0, gsem1, isem0, isem1,
             ssem_lo0, ssem_hi0, ssem_lo1, ssem_hi1):
    cid = lax.axis_index("c")
    sid = lax.axis_index("s")
    wid = cid * NS + sid
    bufs = [
        (is_v0, ir_v0, ir_lo0, ir_hi0, p_v0, q_v0, e_v0, gsem0, isem0,
         ssem_lo0, ssem_hi0),
        (is_v1, ir_v1, ir_lo1, ir_hi1, p_v1, q_v1, e_v1, gsem1, isem1,
         ssem_lo1, ssem_hi1),
    ]

    row0 = sid * RPS
    pltpu.sync_copy(z_hbm.at[pl.ds(row0, RPS)], acc_lo.at[pl.ds(row0, RPS)])
    pltpu.sync_copy(z_hbm.at[pl.ds(row0, RPS)], acc_hi.at[pl.ds(row0, RPS)])

    @pl.when(sid == 0)
    def _():
        pltpu.sync_copy(z_hbm.at[pl.ds(NS * RPS, RTAIL)],
                        acc_lo.at[pl.ds(NS * RPS, RTAIL)])
        pltpu.sync_copy(z_hbm.at[pl.ds(NS * RPS, RTAIL)],
                        acc_hi.at[pl.ds(NS * RPS, RTAIL)])

    pltpu.sync_copy(w2_hbm, w2_v)
    pltpu.sync_copy(b2_hbm, b2_v)
    plsc.subcore_barrier()

    w2s = [w2_v[pl.ds(16 * j, 16)] for j in range(8)]
    b2s = b2_v[...]
    half = jnp.full((16,), HALF, jnp.int32)
    twice = jnp.full((16,), 2 * HALF, jnp.int32)
    trash = jnp.full((16,), TRASH, jnp.int32)
    n_chunks = jnp.where(cid == 0, C0, C1)
    base_w = jnp.where(cid == 0, sid * C0, NS * C0 + sid * C1) * CHUNK

    def fetch_idx(c, buf):
        is_v, ir_v = buf[0], buf[1]
        isem = buf[8]
        base = base_w + c * CHUNK
        pltpu.async_copy(is_hbm.at[pl.ds(base, CHUNK)], is_v, isem)
        pltpu.async_copy(ir_hbm.at[pl.ds(base, CHUNK)], ir_v, isem)

    def fetch(c, buf):
        is_v, ir_v, ir_lo, ir_hi, p_v, q_v, e_v, gsem, isem, _, _ = buf
        base = base_w + c * CHUNK
        pltpu.make_async_copy(is_hbm.at[pl.ds(base, CHUNK)], is_v, isem).wait()
        pltpu.make_async_copy(ir_hbm.at[pl.ds(base, CHUNK)], ir_v, isem).wait()
        pltpu.async_copy(p_hbm.at[is_v], p_v, gsem)
        pltpu.async_copy(q_hbm.at[ir_v], q_v, gsem)
        pltpu.async_copy(eb_hbm.at[pl.ds(base, CHUNK)], e_v, gsem)
        for j in range(CHUNK // 16):
            sl = pl.ds(16 * j, 16)
            iv = ir_v[sl]
            in_lo = iv < half
            in_hi = jnp.logical_and(jnp.logical_not(in_lo), iv < twice)
            ir_lo[sl] = jnp.where(in_lo, iv, trash)
            ir_hi[sl] = jnp.where(in_hi, iv - half, trash)

    def wait_gathers(c, buf):
        is_v, ir_v, _, _, p_v, q_v, e_v, gsem, _, _, _ = buf
        base = base_w + c * CHUNK
        pltpu.make_async_copy(p_hbm.at[is_v], p_v, gsem).wait()
        pltpu.make_async_copy(q_hbm.at[ir_v], q_v, gsem).wait()
        pltpu.make_async_copy(eb_hbm.at[pl.ds(base, CHUNK)], e_v, gsem).wait()

    def compute(buf):
        _, _, _, _, p_v, q_v, e_v, _, _, _, _ = buf

        @plsc.parallel_loop(0, CHUNK, unroll=2)
        def _row(r):
            acc = jnp.zeros((16,), jnp.float32)
            ms = []
            for j in range(8):
                sl = pl.ds(16 * j, 16)
                g = p_v[r, sl] + q_v[r, sl] + e_v[r, sl]
                s = 1.0 / (1.0 + jnp.exp(-g))
                m = g * s
                acc = acc + m * w2s[j]
                ms.append(m)
            t = jnp.sum(acc) + b2s
            w = 1.0 / (1.0 + jnp.exp(-t))
            for j in range(8):
                p_v[r, pl.ds(16 * j, 16)] = ms[j] * w

    def issue_scatters(buf):
        _, _, ir_lo, ir_hi, p_v, _, _, _, _, slo, shi = buf
        pltpu.async_copy(p_v, acc_lo.at[ir_lo], slo, add=True)
        pltpu.async_copy(p_v, acc_hi.at[ir_hi], shi, add=True)

    def wait_scatters(buf):
        _, _, ir_lo, ir_hi, p_v, _, _, _, _, slo, shi = buf
        pltpu.make_async_copy(p_v, acc_lo.at[ir_lo], slo).wait()
        pltpu.make_async_copy(p_v, acc_hi.at[ir_hi], shi).wait()

    fetch_idx(0, bufs[0])
    fetch_idx(1, bufs[1])
    fetch(0, bufs[0])

    @pl.loop(0, n_chunks, step=2)
    def _chunk(c):
        for b in range(2):
            cc = c + b
            cur, nxt = bufs[b], bufs[1 - b]

            @pl.when(cc + 1 < n_chunks)
            def _():
                @pl.when(cc >= 1)
                def _():
                    wait_scatters(nxt)
                fetch(cc + 1, nxt)

            wait_gathers(cc, cur)

            @pl.when(cc + 2 < n_chunks)
            def _():
                fetch_idx(cc + 2, cur)

            compute(cur)
            issue_scatters(cur)

    wait_scatters(bufs[0])
    wait_scatters(bufs[1])

    plsc.subcore_barrier()
    pltpu.sync_copy(acc_lo.at[pl.ds(row0, RPS)],
                    out_hbm.at[cid, pl.ds(row0, RPS)])
    pltpu.sync_copy(acc_hi.at[pl.ds(row0, RPS)],
                    out_hbm.at[cid, pl.ds(HALF + row0, RPS)])

    @pl.when(sid == 0)
    def _():
        pltpu.sync_copy(acc_lo.at[pl.ds(NS * RPS, RTAIL)],
                        out_hbm.at[cid, pl.ds(NS * RPS, RTAIL)])
        pltpu.sync_copy(acc_hi.at[pl.ds(NS * RPS, RTAIL)],
                        out_hbm.at[cid, pl.ds(HALF + NS * RPS, RTAIL)])


def _sc_message_pass(p, q, eb, is_p, ir_p, zeros, w2_flat, b2_vec):
    mesh = plsc.VectorSubcoreMesh(core_axis_name="c", subcore_axis_name="s",
                                  num_cores=NC, num_subcores=NS)
    cp = pltpu.CompilerParams()
    if "needs_layout_passes" in pltpu.CompilerParams.__dataclass_fields__:
        cp = dataclasses.replace(cp, needs_layout_passes=False)
    fn = pl.kernel(
        _sc_body,
        out_type=jax.ShapeDtypeStruct((NC, N, H), jnp.float32),
        mesh=mesh,
        compiler_params=cp,
        scratch_types=(
            [pltpu.VMEM((CHUNK,), jnp.int32)] * 4
            + [pltpu.VMEM((CHUNK, H), jnp.float32)] * 3
            + [pltpu.VMEM((CHUNK,), jnp.int32)] * 4
            + [pltpu.VMEM((CHUNK, H), jnp.float32)] * 3
            + [
                pltpu.VMEM((H,), jnp.float32),
                pltpu.VMEM((16,), jnp.float32),
                pltpu.VMEM_SHARED((HR, H), jnp.float32),
                pltpu.VMEM_SHARED((HR, H), jnp.float32),
            ]
            + [pltpu.SemaphoreType.DMA] * 8
        ),
    )
    return fn(p, q, eb, is_p, ir_p, zeros, w2_flat, b2_vec)


def kernel(x_send, x_rec, edge_attr, W1, b1, W2, b2, index_send, index_rec):
    w1a, w1b, w1c = W1[:H], W1[H:2 * H], W1[2 * H:]
    xs_p = jnp.pad(x_send, ((0, NPAD - N), (0, 0)))
    xr_p = jnp.pad(x_rec, ((0, NPAD - N), (0, 0)))
    ea_p = jnp.pad(edge_attr, ((0, EPAD - E), (0, 0)))
    is_p = jnp.concatenate(
        [index_send, jnp.zeros((EPAD - E,), jnp.int32)])
    ir_p = jnp.concatenate(
        [index_rec, jnp.full((EPAD - E,), N, jnp.int32)])

    p, q = _node_proj(xs_p, xr_p, w1a, w1b)
    eb = _edge_bias(ea_p, w1c, b1.reshape(1, H))

    zeros = jnp.zeros((HALF, H), jnp.float32)
    b2_vec = jnp.broadcast_to(b2, (16,)).astype(jnp.float32)
    parts = _sc_message_pass(p, q, eb, is_p, ir_p, zeros, W2[:, 0], b2_vec)

    return _final_add(x_rec, parts[0], parts[1])


# speedup vs baseline: 1.0977x; 1.0977x over previous
import dataclasses

import jax
import jax.numpy as jnp
from jax import lax
from jax.experimental import pallas as pl
from jax.experimental.pallas import tpu as pltpu
from jax.experimental.pallas import tpu_sc as plsc

N = 10000
E = 320000
H = 128
NUM_INV = 16

NC = 2
NS = 16
NW = NC * NS
CHUNK = 64
NPAD = 10240
EPAD = 327680
CHUNKS_PER_W = EPAD // (NW * CHUNK)
C0 = 200
C1 = 2 * CHUNKS_PER_W - C0
EDGES_PER_W = EPAD // NW
HALF = N // 2
HR = 5008
TRASH = HALF
RPS = 312
RTAIL = HALF - NS * RPS


def _proj_body(xs_ref, xr_ref, wa_ref, wb_ref, p_ref, q_ref):
    dn = (((1,), (0,)), ((), ()))
    p_ref[...] = lax.dot_general(xs_ref[...], wa_ref[...], dn,
                                 precision=lax.Precision.HIGHEST,
                                 preferred_element_type=jnp.float32)
    q_ref[...] = lax.dot_general(xr_ref[...], wb_ref[...], dn,
                                 precision=lax.Precision.HIGHEST,
                                 preferred_element_type=jnp.float32)


def _edge_bias_body(ea_ref, wc_ref, b1_ref, eb_ref):
    dn = (((1,), (0,)), ((), ()))
    eb_ref[...] = lax.dot_general(ea_ref[...], wc_ref[...], dn,
                                  preferred_element_type=jnp.float32) + b1_ref[...]


def _final_body(xr_ref, p0_ref, p1_ref, o_ref):
    o_ref[...] = xr_ref[...] + p0_ref[...] + p1_ref[...]


def _node_proj(xs_p, xr_p, w1a, w1b):
    blk = 1024
    return pl.pallas_call(
        _proj_body,
        grid=(NPAD // blk,),
        in_specs=[
            pl.BlockSpec((blk, H), lambda i: (i, 0)),
            pl.BlockSpec((blk, H), lambda i: (i, 0)),
            pl.BlockSpec((H, H), lambda i: (0, 0)),
            pl.BlockSpec((H, H), lambda i: (0, 0)),
        ],
        out_specs=[
            pl.BlockSpec((blk, H), lambda i: (i, 0)),
            pl.BlockSpec((blk, H), lambda i: (i, 0)),
        ],
        out_shape=[
            jax.ShapeDtypeStruct((NPAD, H), jnp.float32),
            jax.ShapeDtypeStruct((NPAD, H), jnp.float32),
        ],
    )(xs_p, xr_p, w1a, w1b)


def _edge_bias(ea_p, w1c, b1_row):
    blk = 4096
    return pl.pallas_call(
        _edge_bias_body,
        grid=(EPAD // blk,),
        in_specs=[
            pl.BlockSpec((blk, NUM_INV), lambda i: (i, 0)),
            pl.BlockSpec((NUM_INV, H), lambda i: (0, 0)),
            pl.BlockSpec((1, H), lambda i: (0, 0)),
        ],
        out_specs=pl.BlockSpec((blk, H), lambda i: (i, 0)),
        out_shape=jax.ShapeDtypeStruct((EPAD, H), jnp.float32),
    )(ea_p, w1c, b1_row)


def _final_add(x_rec, part0, part1):
    blk = 2000
    return pl.pallas_call(
        _final_body,
        grid=(N // blk,),
        in_specs=[pl.BlockSpec((blk, H), lambda i: (i, 0))] * 3,
        out_specs=pl.BlockSpec((blk, H), lambda i: (i, 0)),
        out_shape=jax.ShapeDtypeStruct((N, H), jnp.float32),
    )(x_rec, part0, part1)


def _sc_body(p_hbm, q_hbm, eb_hbm, is_hbm, ir_hbm, z_hbm, w2_hbm, b2_hbm,
             out_hbm,
             is_v0, ir_v0, ir_lo0, ir_hi0, p_v0, q_v0, e_v0,
             is_v1, ir_v1, ir_lo1, ir_hi1, p_v1, q_v1, e_v1,
             w2_v, b2_v,
             acc_lo, acc_hi,
             gsem0, gsem1, isem0, isem1,
             ssem_lo0, ssem_hi0, ssem_lo1, ssem_hi1):
    cid = lax.axis_index("c")
    sid = lax.axis_index("s")
    wid = cid * NS + sid
    bufs = [
        (is_v0, ir_v0, ir_lo0, ir_hi0, p_v0, q_v0, e_v0, gsem0, isem0,
         ssem_lo0, ssem_hi0),
        (is_v1, ir_v1, ir_lo1, ir_hi1, p_v1, q_v1, e_v1, gsem1, isem1,
         ssem_lo1, ssem_hi1),
    ]

    row0 = sid * RPS
    pltpu.sync_copy(z_hbm.at[pl.ds(row0, RPS)], acc_lo.at[pl.ds(row0, RPS)])
    pltpu.sync_copy(z_hbm.at[pl.ds(row0, RPS)], acc_hi.at[pl.ds(row0, RPS)])

    @pl.when(sid == 0)
    def _():
        pltpu.sync_copy(z_hbm.at[pl.ds(NS * RPS, RTAIL)],
                        acc_lo.at[pl.ds(NS * RPS, RTAIL)])
        pltpu.sync_copy(z_hbm.at[pl.ds(NS * RPS, RTAIL)],
                        acc_hi.at[pl.ds(NS * RPS, RTAIL)])

    pltpu.sync_copy(w2_hbm, w2_v)
    pltpu.sync_copy(b2_hbm, b2_v)
    plsc.subcore_barrier()

    w2s = [w2_v[pl.ds(16 * j, 16)] for j in range(8)]
    b2s = b2_v[...]
    half = jnp.full((16,), HALF, jnp.int32)
    twice = jnp.full((16,), 2 * HALF, jnp.int32)
    trash = jnp.full((16,), TRASH, jnp.int32)
    n_chunks = jnp.where(cid == 0, C0, C1)
    base_w = jnp.where(cid == 0, sid * C0, NS * C0 + sid * C1) * CHUNK

    def fetch_idx(c, buf):
        is_v, ir_v = buf[0], buf[1]
        isem = buf[8]
        base = base_w + c * CHUNK
        pltpu.async_copy(is_hbm.at[pl.ds(base, CHUNK)], is_v, isem)
        pltpu.async_copy(ir_hbm.at[pl.ds(base, CHUNK)], ir_v, isem)

    def fetch(c, buf):
        is_v, ir_v, ir_lo, ir_hi, p_v, q_v, e_v, gsem, isem, _, _ = buf
        base = base_w + c * CHUNK
        pltpu.make_async_copy(is_hbm.at[pl.ds(base, CHUNK)], is_v, isem).wait()
        pltpu.make_async_copy(ir_hbm.at[pl.ds(base, CHUNK)], ir_v, isem).wait()
        pltpu.async_copy(p_hbm.at[is_v], p_v, gsem)
        pltpu.async_copy(q_hbm.at[ir_v], q_v, gsem)
        pltpu.async_copy(eb_hbm.at[pl.ds(base, CHUNK)], e_v, gsem)
        for j in range(CHUNK // 16):
            sl = pl.ds(16 * j, 16)
            iv = ir_v[sl]
            in_lo = iv < half
            in_hi = jnp.logical_and(jnp.logical_not(in_lo), iv < twice)
            ir_lo[sl] = jnp.where(in_lo, iv, trash)
            ir_hi[sl] = jnp.where(in_hi, iv - half, trash)

    def wait_gathers(c, buf):
        is_v, ir_v, _, _, p_v, q_v, e_v, gsem, _, _, _ = buf
        base = base_w + c * CHUNK
        pltpu.make_async_copy(p_hbm.at[is_v], p_v, gsem).wait()
        pltpu.make_async_copy(q_hbm.at[ir_v], q_v, gsem).wait()
        pltpu.make_async_copy(eb_hbm.at[pl.ds(base, CHUNK)], e_v, gsem).wait()

    def compute(buf):
        _, _, _, _, p_v, q_v, e_v, _, _, _, _ = buf

        @plsc.parallel_loop(0, CHUNK, unroll=2)
        def _row(r):
            acc = jnp.zeros((16,), jnp.float32)
            ms = []
            for j in range(8):
                sl = pl.ds(16 * j, 16)
                g = p_v[r, sl] + q_v[r, sl] + e_v[r, sl]
                s = 1.0 / (1.0 + jnp.exp(-g))
                m = g * s
                acc = acc + m * w2s[j]
                ms.append(m)
            t = jnp.sum(acc) + b2s
            w = 1.0 / (1.0 + jnp.exp(-t))
            for j in range(8):
                p_v[r, pl.ds(16 * j, 16)] = ms[j] * w

    def issue_scatters(buf):
        _, _, ir_lo, ir_hi, p_v, _, _, _, _, slo, shi = buf
        pltpu.async_copy(p_v, acc_lo.at[ir_lo], slo, add=True)
        pltpu.async_copy(p_v, acc_hi.at[ir_hi], shi, add=True)

    def wait_scatters(buf):
        _, _, ir_lo, ir_hi, p_v, _, _, _, _, slo, shi = buf
        pltpu.make_async_copy(p_v, acc_lo.at[ir_lo], slo).wait()
        pltpu.make_async_copy(p_v, acc_hi.at[ir_hi], shi).wait()

    fetch_idx(0, bufs[0])
    fetch_idx(1, bufs[1])
    fetch(0, bufs[0])

    @pl.loop(0, n_chunks, step=2)
    def _chunk(c):
        for b in range(2):
            cc = c + b
            cur, nxt = bufs[b], bufs[1 - b]

            @pl.when(cc + 1 < n_chunks)
            def _():
                @pl.when(cc >= 1)
                def _():
                    wait_scatters(nxt)
                fetch(cc + 1, nxt)

            wait_gathers(cc, cur)

            @pl.when(cc + 2 < n_chunks)
            def _():
                fetch_idx(cc + 2, cur)

            compute(cur)
            issue_scatters(cur)

    wait_scatters(bufs[0])
    wait_scatters(bufs[1])

    plsc.subcore_barrier()
    pltpu.sync_copy(acc_lo.at[pl.ds(row0, RPS)],
                    out_hbm.at[cid, pl.ds(row0, RPS)])
    pltpu.sync_copy(acc_hi.at[pl.ds(row0, RPS)],
                    out_hbm.at[cid, pl.ds(HALF + row0, RPS)])

    @pl.when(sid == 0)
    def _():
        pltpu.sync_copy(acc_lo.at[pl.ds(NS * RPS, RTAIL)],
                        out_hbm.at[cid, pl.ds(NS * RPS, RTAIL)])
        pltpu.sync_copy(acc_hi.at[pl.ds(NS * RPS, RTAIL)],
                        out_hbm.at[cid, pl.ds(HALF + NS * RPS, RTAIL)])


def _sc_message_pass(p, q, eb, is_p, ir_p, zeros, w2_flat, b2_vec):
    mesh = plsc.VectorSubcoreMesh(core_axis_name="c", subcore_axis_name="s",
                                  num_cores=NC, num_subcores=NS)
    cp = pltpu.CompilerParams()
    if "needs_layout_passes" in pltpu.CompilerParams.__dataclass_fields__:
        cp = dataclasses.replace(cp, needs_layout_passes=False)
    fn = pl.kernel(
        _sc_body,
        out_type=jax.ShapeDtypeStruct((NC, N, H), jnp.float32),
        mesh=mesh,
        compiler_params=cp,
        scratch_types=(
            [pltpu.VMEM((CHUNK,), jnp.int32)] * 4
            + [pltpu.VMEM((CHUNK, H), jnp.float32)] * 3
            + [pltpu.VMEM((CHUNK,), jnp.int32)] * 4
            + [pltpu.VMEM((CHUNK, H), jnp.float32)] * 3
            + [
                pltpu.VMEM((H,), jnp.float32),
                pltpu.VMEM((16,), jnp.float32),
                pltpu.VMEM_SHARED((HR, H), jnp.float32),
                pltpu.VMEM_SHARED((HR, H), jnp.float32),
            ]
            + [pltpu.SemaphoreType.DMA] * 8
        ),
    )
    return fn(p, q, eb, is_p, ir_p, zeros, w2_flat, b2_vec)


def kernel(x_send, x_rec, edge_attr, W1, b1, W2, b2, index_send, index_rec):
    w1a, w1b, w1c = W1[:H], W1[H:2 * H], W1[2 * H:]
    xs_p = jnp.pad(x_send, ((0, NPAD - N), (0, 0)))
    xr_p = jnp.pad(x_rec, ((0, NPAD - N), (0, 0)))
    ea_p = jnp.pad(edge_attr, ((0, EPAD - E), (0, 0)))
    is_p = jnp.concatenate(
        [index_send, jnp.zeros((EPAD - E,), jnp.int32)])
    ir_p = jnp.concatenate(
        [index_rec, jnp.full((EPAD - E,), N, jnp.int32)])

    p, q = _node_proj(xs_p, xr_p, w1a, w1b)
    eb = _edge_bias(ea_p, w1c, b1.reshape(1, H))

    zeros = jnp.zeros((HALF, H), jnp.float32)
    b2_vec = jnp.broadcast_to(b2, (16,)).astype(jnp.float32)
    parts = _sc_message_pass(p, q, eb, is_p, ir_p, zeros, W2[:, 0], b2_vec)

    return _final_add(x_rec, parts[0], parts[1])


# speedup vs baseline: 1.1009x; 1.0030x over previous
import dataclasses

import jax
import jax.numpy as jnp
from jax import lax
from jax.experimental import pallas as pl
from jax.experimental.pallas import tpu as pltpu
from jax.experimental.pallas import tpu_sc as plsc

N = 10000
E = 320000
H = 128
NUM_INV = 16

NC = 2
NS = 16
NW = NC * NS
CHUNK = 64
NPAD = 10240
EPAD = 327680
CHUNKS_PER_W = EPAD // (NW * CHUNK)
C0 = 204
C1 = 2 * CHUNKS_PER_W - C0
EDGES_PER_W = EPAD // NW
HALF = N // 2
HR = 5008
TRASH = HALF
RPS = 312
RTAIL = HALF - NS * RPS


def _proj_body(xs_ref, xr_ref, wa_ref, wb_ref, p_ref, q_ref):
    dn = (((1,), (0,)), ((), ()))
    p_ref[...] = lax.dot_general(xs_ref[...], wa_ref[...], dn,
                                 precision=lax.Precision.HIGHEST,
                                 preferred_element_type=jnp.float32)
    q_ref[...] = lax.dot_general(xr_ref[...], wb_ref[...], dn,
                                 precision=lax.Precision.HIGHEST,
                                 preferred_element_type=jnp.float32)


def _edge_bias_body(ea_ref, wc_ref, b1_ref, eb_ref):
    dn = (((1,), (0,)), ((), ()))
    eb_ref[...] = lax.dot_general(ea_ref[...], wc_ref[...], dn,
                                  preferred_element_type=jnp.float32) + b1_ref[...]


def _final_body(xr_ref, p0_ref, p1_ref, o_ref):
    o_ref[...] = xr_ref[...] + p0_ref[...] + p1_ref[...]


def _node_proj(xs_p, xr_p, w1a, w1b):
    blk = 1024
    return pl.pallas_call(
        _proj_body,
        grid=(NPAD // blk,),
        in_specs=[
            pl.BlockSpec((blk, H), lambda i: (i, 0)),
            pl.BlockSpec((blk, H), lambda i: (i, 0)),
            pl.BlockSpec((H, H), lambda i: (0, 0)),
            pl.BlockSpec((H, H), lambda i: (0, 0)),
        ],
        out_specs=[
            pl.BlockSpec((blk, H), lambda i: (i, 0)),
            pl.BlockSpec((blk, H), lambda i: (i, 0)),
        ],
        out_shape=[
            jax.ShapeDtypeStruct((NPAD, H), jnp.float32),
            jax.ShapeDtypeStruct((NPAD, H), jnp.float32),
        ],
    )(xs_p, xr_p, w1a, w1b)


def _edge_bias(ea_p, w1c, b1_row):
    blk = 4096
    return pl.pallas_call(
        _edge_bias_body,
        grid=(EPAD // blk,),
        in_specs=[
            pl.BlockSpec((blk, NUM_INV), lambda i: (i, 0)),
            pl.BlockSpec((NUM_INV, H), lambda i: (0, 0)),
            pl.BlockSpec((1, H), lambda i: (0, 0)),
        ],
        out_specs=pl.BlockSpec((blk, H), lambda i: (i, 0)),
        out_shape=jax.ShapeDtypeStruct((EPAD, H), jnp.float32),
    )(ea_p, w1c, b1_row)


def _final_add(x_rec, part0, part1):
    blk = 2000
    return pl.pallas_call(
        _final_body,
        grid=(N // blk,),
        in_specs=[pl.BlockSpec((blk, H), lambda i: (i, 0))] * 3,
        out_specs=pl.BlockSpec((blk, H), lambda i: (i, 0)),
        out_shape=jax.ShapeDtypeStruct((N, H), jnp.float32),
    )(x_rec, part0, part1)


def _sc_body(p_hbm, q_hbm, eb_hbm, is_hbm, ir_hbm, z_hbm, w2_hbm, b2_hbm,
             out_hbm,
             is_v0, ir_v0, ir_lo0, ir_hi0, p_v0, q_v0, e_v0,
             is_v1, ir_v1, ir_lo1, ir_hi1, p_v1, q_v1, e_v1,
             w2_v, b2_v,
             acc_lo, acc_hi,
             gsem0, gsem1, isem0, isem1,
             ssem_lo0, ssem_hi0, ssem_lo1, ssem_hi1):
    cid = lax.axis_index("c")
    sid = lax.axis_index("s")
    wid = cid * NS + sid
    bufs = [
        (is_v0, ir_v0, ir_lo0, ir_hi0, p_v0, q_v0, e_v0, gsem0, isem0,
         ssem_lo0, ssem_hi0),
        (is_v1, ir_v1, ir_lo1, ir_hi1, p_v1, q_v1, e_v1, gsem1, isem1,
         ssem_lo1, ssem_hi1),
    ]

    row0 = sid * RPS
    pltpu.sync_copy(z_hbm.at[pl.ds(row0, RPS)], acc_lo.at[pl.ds(row0, RPS)])
    pltpu.sync_copy(z_hbm.at[pl.ds(row0, RPS)], acc_hi.at[pl.ds(row0, RPS)])

    @pl.when(sid == 0)
    def _():
        pltpu.sync_copy(z_hbm.at[pl.ds(NS * RPS, RTAIL)],
                        acc_lo.at[pl.ds(NS * RPS, RTAIL)])
        pltpu.sync_copy(z_hbm.at[pl.ds(NS * RPS, RTAIL)],
                        acc_hi.at[pl.ds(NS * RPS, RTAIL)])

    pltpu.sync_copy(w2_hbm, w2_v)
    pltpu.sync_copy(b2_hbm, b2_v)
    plsc.subcore_barrier()

    w2s = [w2_v[pl.ds(16 * j, 16)] for j in range(8)]
    b2s = b2_v[...]
    half = jnp.full((16,), HALF, jnp.int32)
    twice = jnp.full((16,), 2 * HALF, jnp.int32)
    trash = jnp.full((16,), TRASH, jnp.int32)
    n_chunks = jnp.where(cid == 0, C0, C1)
    base_w = jnp.where(cid == 0, sid * C0, NS * C0 + sid * C1) * CHUNK

    def fetch_idx(c, buf):
        is_v, ir_v = buf[0], buf[1]
        isem = buf[8]
        base = base_w + c * CHUNK
        pltpu.async_copy(is_hbm.at[pl.ds(base, CHUNK)], is_v, isem)
        pltpu.async_copy(ir_hbm.at[pl.ds(base, CHUNK)], ir_v, isem)

    def fetch(c, buf):
        is_v, ir_v, ir_lo, ir_hi, p_v, q_v, e_v, gsem, isem, _, _ = buf
        base = base_w + c * CHUNK
        pltpu.make_async_copy(is_hbm.at[pl.ds(base, CHUNK)], is_v, isem).wait()
        pltpu.make_async_copy(ir_hbm.at[pl.ds(base, CHUNK)], ir_v, isem).wait()
        pltpu.async_copy(p_hbm.at[is_v], p_v, gsem)
        pltpu.async_copy(q_hbm.at[ir_v], q_v, gsem)
        pltpu.async_copy(eb_hbm.at[pl.ds(base, CHUNK)], e_v, gsem)
        for j in range(CHUNK // 16):
            sl = pl.ds(16 * j, 16)
            iv = ir_v[sl]
            in_lo = iv < half
            in_hi = jnp.logical_and(jnp.logical_not(in_lo), iv < twice)
            ir_lo[sl] = jnp.where(in_lo, iv, trash)
            ir_hi[sl] = jnp.where(in_hi, iv - half, trash)

    def wait_gathers(c, buf):
        is_v, ir_v, _, _, p_v, q_v, e_v, gsem, _, _, _ = buf
        base = base_w + c * CHUNK
        pltpu.make_async_copy(p_hbm.at[is_v], p_v, gsem).wait()
        pltpu.make_async_copy(q_hbm.at[ir_v], q_v, gsem).wait()
        pltpu.make_async_copy(eb_hbm.at[pl.ds(base, CHUNK)], e_v, gsem).wait()

    def compute(buf):
        _, _, _, _, p_v, q_v, e_v, _, _, _, _ = buf

        @plsc.parallel_loop(0, CHUNK, unroll=2)
        def _row(r):
            acc = jnp.zeros((16,), jnp.float32)
            ms = []
            for j in range(8):
                sl = pl.ds(16 * j, 16)
                g = p_v[r, sl] + q_v[r, sl] + e_v[r, sl]
                s = 1.0 / (1.0 + jnp.exp(-g))
                m = g * s
                acc = acc + m * w2s[j]
                ms.append(m)
            t = jnp.sum(acc) + b2s
            w = 1.0 / (1.0 + jnp.exp(-t))
            for j in range(8):
                p_v[r, pl.ds(16 * j, 16)] = ms[j] * w

    def issue_scatters(buf):
        _, _, ir_lo, ir_hi, p_v, _, _, _, _, slo, shi = buf
        pltpu.async_copy(p_v, acc_lo.at[ir_lo], slo, add=True)
        pltpu.async_copy(p_v, acc_hi.at[ir_hi], shi, add=True)

    def wait_scatters(buf):
        _, _, ir_lo, ir_hi, p_v, _, _, _, _, slo, shi = buf
        pltpu.make_async_copy(p_v, acc_lo.at[ir_lo], slo).wait()
        pltpu.make_async_copy(p_v, acc_hi.at[ir_hi], shi).wait()

    fetch_idx(0, bufs[0])
    fetch_idx(1, bufs[1])
    fetch(0, bufs[0])

    @pl.loop(0, n_chunks, step=2)
    def _chunk(c):
        for b in range(2):
            cc = c + b
            cur, nxt = bufs[b], bufs[1 - b]

            @pl.when(cc + 1 < n_chunks)
            def _():
                @pl.when(cc >= 1)
                def _():
                    wait_scatters(nxt)
                fetch(cc + 1, nxt)

            wait_gathers(cc, cur)

            @pl.when(cc + 2 < n_chunks)
            def _():
                fetch_idx(cc + 2, cur)

            compute(cur)
            issue_scatters(cur)

    wait_scatters(bufs[0])
    wait_scatters(bufs[1])

    plsc.subcore_barrier()
    pltpu.sync_copy(acc_lo.at[pl.ds(row0, RPS)],
                    out_hbm.at[cid, pl.ds(row0, RPS)])
    pltpu.sync_copy(acc_hi.at[pl.ds(row0, RPS)],
                    out_hbm.at[cid, pl.ds(HALF + row0, RPS)])

    @pl.when(sid == 0)
    def _():
        pltpu.sync_copy(acc_lo.at[pl.ds(NS * RPS, RTAIL)],
                        out_hbm.at[cid, pl.ds(NS * RPS, RTAIL)])
        pltpu.sync_copy(acc_hi.at[pl.ds(NS * RPS, RTAIL)],
                        out_hbm.at[cid, pl.ds(HALF + NS * RPS, RTAIL)])


def _sc_message_pass(p, q, eb, is_p, ir_p, zeros, w2_flat, b2_vec):
    mesh = plsc.VectorSubcoreMesh(core_axis_name="c", subcore_axis_name="s",
                                  num_cores=NC, num_subcores=NS)
    cp = pltpu.CompilerParams()
    if "needs_layout_passes" in pltpu.CompilerParams.__dataclass_fields__:
        cp = dataclasses.replace(cp, needs_layout_passes=False)
    fn = pl.kernel(
        _sc_body,
        out_type=jax.ShapeDtypeStruct((NC, N, H), jnp.float32),
        mesh=mesh,
        compiler_params=cp,
        scratch_types=(
            [pltpu.VMEM((CHUNK,), jnp.int32)] * 4
            + [pltpu.VMEM((CHUNK, H), jnp.float32)] * 3
            + [pltpu.VMEM((CHUNK,), jnp.int32)] * 4
            + [pltpu.VMEM((CHUNK, H), jnp.float32)] * 3
            + [
                pltpu.VMEM((H,), jnp.float32),
                pltpu.VMEM((16,), jnp.float32),
                pltpu.VMEM_SHARED((HR, H), jnp.float32),
                pltpu.VMEM_SHARED((HR, H), jnp.float32),
            ]
            + [pltpu.SemaphoreType.DMA] * 8
        ),
    )
    return fn(p, q, eb, is_p, ir_p, zeros, w2_flat, b2_vec)


def kernel(x_send, x_rec, edge_attr, W1, b1, W2, b2, index_send, index_rec):
    w1a, w1b, w1c = W1[:H], W1[H:2 * H], W1[2 * H:]
    xs_p = jnp.pad(x_send, ((0, NPAD - N), (0, 0)))
    xr_p = jnp.pad(x_rec, ((0, NPAD - N), (0, 0)))
    ea_p = jnp.pad(edge_attr, ((0, EPAD - E), (0, 0)))
    is_p = jnp.concatenate(
        [index_send, jnp.zeros((EPAD - E,), jnp.int32)])
    ir_p = jnp.concatenate(
        [index_rec, jnp.full((EPAD - E,), N, jnp.int32)])

    p, q = _node_proj(xs_p, xr_p, w1a, w1b)
    eb = _edge_bias(ea_p, w1c, b1.reshape(1, H))

    zeros = jnp.zeros((HALF, H), jnp.float32)
    b2_vec = jnp.broadcast_to(b2, (16,)).astype(jnp.float32)
    parts = _sc_message_pass(p, q, eb, is_p, ir_p, zeros, W2[:, 0], b2_vec)

    return _final_add(x_rec, parts[0], parts[1])
